# Initial kernel scaffold; baseline (speedup 1.0000x reference)
import jax
import jax.numpy as jnp
from jax.experimental import pallas as pl


def kernel(x, edge_index):
    raise NotImplementedError("write your pallas kernel here")


# speedup vs baseline: 7.1167x; 7.1167x over previous
import functools

import jax
import jax.numpy as jnp
from jax import lax
from jax.experimental import pallas as pl
from jax.experimental.pallas import tpu as pltpu
from jax.experimental.pallas import tpu_sc as plsc

N = 10000
E = 320000
D = 128
NUM_LAYERS = 3

NC = 2
NS = 16
NW = NC * NS

CHUNK = 128
CPT = 79
EPT = CHUNK * CPT
EPAD = NW * EPT

NPAD = 10240
RPT = NPAD // NS

@functools.cache
def _mesh():
    return plsc.VectorSubcoreMesh(
        core_axis_name="c", subcore_axis_name="s", num_cores=NC, num_subcores=NS
    )


def _deg_body(dst_hbm, degp_hbm, didx, ones_v, zbuf, dacc):
    cid = lax.axis_index("c")
    sid = lax.axis_index("s")
    wid = cid * NS + sid
    base = wid * EPT

    z16 = jnp.zeros((16,), jnp.float32)
    o16 = jnp.ones((16,), jnp.float32)
    for j in range(CHUNK // 16):
        ones_v[pl.ds(j * 16, 16)] = o16
    for j in range(RPT // 16):
        zbuf[pl.ds(j * 16, 16)] = z16

    pltpu.sync_copy(zbuf, dacc.at[pl.ds(sid * RPT, RPT)])
    plsc.subcore_barrier()

    def step(c, carry):
        off = base + c * CHUNK
        pltpu.sync_copy(dst_hbm.at[pl.ds(off, CHUNK)], didx)
        pltpu.sync_copy(ones_v, dacc.at[didx], add=True)
        return carry

    lax.fori_loop(0, CPT, step, 0)
    plsc.subcore_barrier()
    pltpu.sync_copy(dacc.at[pl.ds(sid * RPT, RPT)],
                    degp_hbm.at[cid, pl.ds(sid * RPT, RPT)])


@jax.jit
def _deg_call(dst_p):
    return pl.kernel(
        _deg_body,
        out_type=jax.ShapeDtypeStruct((NC, NPAD), jnp.float32),
        mesh=_mesh(),
        scratch_types=[
            pltpu.VMEM((CHUNK,), jnp.int32),
            pltpu.VMEM((CHUNK,), jnp.float32),
            pltpu.VMEM((RPT,), jnp.float32),
            pltpu.VMEM_SHARED((NPAD,), jnp.float32),
        ],
    )(dst_p)


def _layer_body(t_hbm, src_hbm, dst_hbm, sp_hbm, sidx, didx, rows, acc, sem):
    cid = lax.axis_index("c")
    sid = lax.axis_index("s")
    wid = cid * NS + sid
    base = wid * EPT

    z16 = jnp.zeros((16,), jnp.float32)

    def zrow(r, carry):
        for j in range(D // 16):
            rows[r, pl.ds(j * 16, 16)] = z16
        return carry

    lax.fori_loop(0, CHUNK, zrow, 0)
    for b in range(RPT // CHUNK):
        pltpu.sync_copy(rows, acc.at[pl.ds(sid * RPT + b * CHUNK, CHUNK)])
    plsc.subcore_barrier()

    def step(c, carry):
        off = base + c * CHUNK
        pltpu.sync_copy(src_hbm.at[pl.ds(off, CHUNK)], sidx)
        pltpu.async_copy(t_hbm.at[sidx], rows, sem).wait()
        pltpu.sync_copy(dst_hbm.at[pl.ds(off, CHUNK)], didx)
        pltpu.sync_copy(rows, acc.at[didx], add=True)
        return carry

    lax.fori_loop(0, CPT, step, 0)
    plsc.subcore_barrier()
    pltpu.sync_copy(acc.at[pl.ds(sid * RPT, RPT)],
                    sp_hbm.at[cid, pl.ds(sid * RPT, RPT)])


@jax.jit
def _layer_call(t, src_p, dst_p):
    return pl.kernel(
        _layer_body,
        out_type=jax.ShapeDtypeStruct((NC, NPAD, D), jnp.float32),
        mesh=_mesh(),
        scratch_types=[
            pltpu.VMEM((CHUNK,), jnp.int32),
            pltpu.VMEM((CHUNK,), jnp.int32),
            pltpu.VMEM((CHUNK, D), jnp.float32),
            pltpu.VMEM_SHARED((NPAD, D), jnp.float32),
            pltpu.SemaphoreType.DMA,
        ],
    )(t, src_p, dst_p)


def _scales_body(degp_ref, dis_ref, d2_ref):
    deg = degp_ref[0] + degp_ref[1]
    dis = jnp.where(deg > 0, lax.rsqrt(deg), 0.0)
    dis_ref[...] = dis
    d2_ref[...] = dis * dis


@jax.jit
def _scales_call(degp3):
    return pl.pallas_call(
        _scales_body,
        out_shape=(
            jax.ShapeDtypeStruct((NPAD // 128, 128), jnp.float32),
            jax.ShapeDtypeStruct((NPAD // 128, 128), jnp.float32),
        ),
    )(degp3)


def _rowscale_body(x_ref, s_ref, o_ref):
    o_ref[...] = x_ref[...] * s_ref[...]


@jax.jit
def _rowscale_call(x_p, dis_c):
    blk = 2048
    return pl.pallas_call(
        _rowscale_body,
        grid=(NPAD // blk,),
        in_specs=[
            pl.BlockSpec((blk, D), lambda i: (i, 0)),
            pl.BlockSpec((blk, 1), lambda i: (i, 0)),
        ],
        out_specs=pl.BlockSpec((blk, D), lambda i: (i, 0)),
        out_shape=jax.ShapeDtypeStruct((NPAD, D), jnp.float32),
    )(x_p, dis_c)


def _mid_body(sp_ref, d2_ref, prev_ref, t_ref, ssum_ref):
    s = sp_ref[0] + sp_ref[1]
    ssum_ref[...] = prev_ref[...] + s
    t_ref[...] = s * d2_ref[...]


@jax.jit
def _mid_call(sp, d2_c, prev):
    blk = 2048
    return pl.pallas_call(
        _mid_body,
        grid=(NPAD // blk,),
        in_specs=[
            pl.BlockSpec((NC, blk, D), lambda i: (0, i, 0)),
            pl.BlockSpec((blk, 1), lambda i: (i, 0)),
            pl.BlockSpec((blk, D), lambda i: (i, 0)),
        ],
        out_specs=[
            pl.BlockSpec((blk, D), lambda i: (i, 0)),
            pl.BlockSpec((blk, D), lambda i: (i, 0)),
        ],
        out_shape=(
            jax.ShapeDtypeStruct((NPAD, D), jnp.float32),
            jax.ShapeDtypeStruct((NPAD, D), jnp.float32),
        ),
    )(sp, d2_c, prev)


def _final_body(sp_ref, dis_ref, prev_ref, o_ref):
    s = sp_ref[0] + sp_ref[1]
    alpha = 1.0 / (1.0 + NUM_LAYERS)
    o_ref[...] = (prev_ref[...] + s) * (dis_ref[...] * alpha)


@jax.jit
def _final_call(sp, dis_c, prev):
    blk = 2048
    return pl.pallas_call(
        _final_body,
        grid=(NPAD // blk,),
        in_specs=[
            pl.BlockSpec((NC, blk, D), lambda i: (0, i, 0)),
            pl.BlockSpec((blk, 1), lambda i: (i, 0)),
            pl.BlockSpec((blk, D), lambda i: (i, 0)),
        ],
        out_specs=pl.BlockSpec((blk, D), lambda i: (i, 0)),
        out_shape=jax.ShapeDtypeStruct((NPAD, D), jnp.float32),
    )(sp, dis_c, prev)


def kernel(x, edge_index):
    src = edge_index[0]
    dst = edge_index[1]
    pad = EPAD - E
    fill = jnp.full((pad,), NPAD - 1, jnp.int32)
    src_p = jnp.concatenate([src.astype(jnp.int32), fill])
    dst_p = jnp.concatenate([dst.astype(jnp.int32), fill])
    x_p = jnp.zeros((NPAD, D), jnp.float32).at[:N].set(x)

    degp = _deg_call(dst_p)
    dis80, d280 = _scales_call(degp.reshape(NC, NPAD // 128, 128))
    dis_c = dis80.reshape(NPAD, 1)
    d2_c = d280.reshape(NPAD, 1)

    t = _rowscale_call(x_p, dis_c)
    prev = jnp.zeros((NPAD, D), jnp.float32)
    for layer in range(NUM_LAYERS):
        sp = _layer_call(t, src_p, dst_p)
        if layer < NUM_LAYERS - 1:
            t, prev = _mid_call(sp, d2_c, prev)
        else:
            out_p = _final_call(sp, dis_c, prev)
    return out_p[:N]
